# Initial kernel scaffold; baseline (speedup 1.0000x reference)
import jax
import jax.numpy as jnp
from jax.experimental import pallas as pl


def kernel(xyz, t, volume):
    raise NotImplementedError("write your pallas kernel here")


# speedup vs baseline: 1.1583x; 1.1583x over previous
import functools

import jax
import jax.numpy as jnp
from jax import lax
from jax.experimental import pallas as pl
from jax.experimental.pallas import tpu as pltpu
from jax.experimental.pallas import tpu_sc as plsc

N = 2097152
NC = 2
NS = 16
NW = NC * NS
PPW = N // NW
CH = 2048
NCHUNK = PPW // CH
L = 16
NV = CH // L

_OFFS = (0, 1, 256, 257, 65536, 65537, 65792, 65793)


def _build():
    mesh = plsc.VectorSubcoreMesh(core_axis_name="c", subcore_axis_name="s")

    @functools.partial(
        pl.kernel,
        mesh=mesh,
        out_type=jax.ShapeDtypeStruct((N,), jnp.float32),
        scratch_types=[
            pltpu.VMEM((CH,), jnp.float32),
            pltpu.VMEM((CH,), jnp.float32),
            pltpu.VMEM((CH,), jnp.float32),
            pltpu.VMEM((CH,), jnp.float32),
            pltpu.VMEM((CH,), jnp.float32),
            pltpu.VMEM((CH,), jnp.float32),
            pltpu.VMEM((8 * CH,), jnp.int32),
            pltpu.VMEM((8 * CH,), jnp.float32),
            pltpu.VMEM((CH,), jnp.float32),
            pltpu.SemaphoreType.DMA,
        ],
    )
    def k(xs_h, ys_h, zs_h, vol_h, out_h,
          xv, yv, zv, wxv, wyv, wzv, idxv, valv, outv, sem):
        wid = lax.axis_index("s") * NC + lax.axis_index("c")
        base_w = wid * PPW

        def chunk(g, carry):
            base = base_w + g * CH
            pltpu.sync_copy(xs_h.at[pl.ds(base, CH)], xv)
            pltpu.sync_copy(ys_h.at[pl.ds(base, CH)], yv)
            pltpu.sync_copy(zs_h.at[pl.ds(base, CH)], zv)

            def compute(v, c2):
                s = pl.ds(v * L, L)

                def prep(pv, wv):
                    f = ((pv[s] * 2.0 - 1.0) + 1.0) * (0.5 * 255.0)
                    i = f.astype(jnp.int32)
                    wv[s] = f - i.astype(jnp.float32)
                    return jnp.minimum(i, 254)

                ixi = prep(xv, wxv)
                iyi = prep(yv, wyv)
                izi = prep(zv, wzv)
                flat = izi * 65536 + iyi * 256 + ixi
                for kk in range(8):
                    idxv[pl.ds(kk * CH + v * L, L)] = flat + _OFFS[kk]
                return c2

            lax.fori_loop(0, NV, compute, 0)
            pltpu.async_copy(vol_h.at[idxv], valv, sem).wait()

            def combine(v, c2):
                s = pl.ds(v * L, L)
                wx = wxv[s]
                wy = wyv[s]
                wz = wzv[s]

                def val(kk):
                    return valv[pl.ds(kk * CH + v * L, L)]

                c00 = val(0) + wx * (val(1) - val(0))
                c01 = val(2) + wx * (val(3) - val(2))
                c10 = val(4) + wx * (val(5) - val(4))
                c11 = val(6) + wx * (val(7) - val(6))
                c0 = c00 + wy * (c01 - c00)
                c1 = c10 + wy * (c11 - c10)
                outv[s] = c0 + wz * (c1 - c0)
                return c2

            lax.fori_loop(0, NV, combine, 0)
            pltpu.sync_copy(outv, out_h.at[pl.ds(base, CH)])
            return carry

        lax.fori_loop(0, NCHUNK, chunk, 0)

    return k


_SAMPLE = _build()


def kernel(xyz, t, volume):
    del t
    xs = xyz[:, 0]
    ys = xyz[:, 1]
    zs = xyz[:, 2]
    vol = volume.reshape(-1)
    out = _SAMPLE(xs, ys, zs, vol)
    return out.reshape(-1, 1)


# speedup vs baseline: 1.6299x; 1.4071x over previous
import functools

import jax
import jax.numpy as jnp
from jax import lax
from jax.experimental import pallas as pl
from jax.experimental.pallas import tpu as pltpu
from jax.experimental.pallas import tpu_sc as plsc

N = 2097152
NC = 2
NS = 16
NW = NC * NS
PPW = N // NW
CH = 2048
NCHUNK = PPW // CH
NPAIR = NCHUNK // 2
L = 16
NV = CH // L

_OFFS = (0, 1, 256, 257, 65536, 65537, 65792, 65793)


def _build():
    mesh = plsc.VectorSubcoreMesh(core_axis_name="c", subcore_axis_name="s")

    buf = lambda n: pltpu.VMEM((n,), jnp.float32)

    @functools.partial(
        pl.kernel,
        mesh=mesh,
        out_type=jax.ShapeDtypeStruct((N,), jnp.float32),
        scratch_types=[
            buf(CH), buf(CH), buf(CH),
            buf(CH), buf(CH), buf(CH),
            buf(CH), buf(CH), buf(CH),
            pltpu.VMEM((8 * CH,), jnp.int32),
            pltpu.VMEM((8 * CH,), jnp.int32),
            pltpu.VMEM((8 * CH,), jnp.float32),
            pltpu.VMEM((8 * CH,), jnp.float32),
            buf(CH),
            pltpu.SemaphoreType.DMA,
            pltpu.SemaphoreType.DMA,
        ],
    )
    def k(xs_h, ys_h, zs_h, vol_h, out_h,
          xv, yv, zv, wxa, wya, wza, wxb, wyb, wzb,
          idxa, idxb, vala, valb, outv, sema, semb):
        wid = lax.axis_index("s") * NC + lax.axis_index("c")
        base_w = wid * PPW

        def compute(g, idxv, wxv, wyv, wzv):
            base = base_w + g * CH
            pltpu.sync_copy(xs_h.at[pl.ds(base, CH)], xv)
            pltpu.sync_copy(ys_h.at[pl.ds(base, CH)], yv)
            pltpu.sync_copy(zs_h.at[pl.ds(base, CH)], zv)

            def body(v, c2):
                s = pl.ds(v * L, L)

                def prep(pv, wv):
                    f = ((pv[s] * 2.0 - 1.0) + 1.0) * 0.5 * 255.0
                    i = f.astype(jnp.int32)
                    wv[s] = f - i.astype(jnp.float32)
                    return jnp.minimum(i, 254)

                ixi = prep(xv, wxv)
                iyi = prep(yv, wyv)
                izi = prep(zv, wzv)
                flat = izi * 65536 + iyi * 256 + ixi
                for kk in range(8):
                    idxv[pl.ds(kk * CH + v * L, L)] = flat + _OFFS[kk]
                return c2

            lax.fori_loop(0, NV, body, 0)

        def start(idxv, valv, sem):
            pltpu.make_async_copy(vol_h.at[idxv], valv, sem).start()

        def wait(idxv, valv, sem):
            pltpu.make_async_copy(vol_h.at[idxv], valv, sem).wait()

        def combine(g, valv, wxv, wyv, wzv):
            def body(v, c2):
                s = pl.ds(v * L, L)
                wx = wxv[s]
                wy = wyv[s]
                wz = wzv[s]

                def val(kk):
                    return valv[pl.ds(kk * CH + v * L, L)]

                c00 = val(0) + wx * (val(1) - val(0))
                c01 = val(2) + wx * (val(3) - val(2))
                c10 = val(4) + wx * (val(5) - val(4))
                c11 = val(6) + wx * (val(7) - val(6))
                c0 = c00 + wy * (c01 - c00)
                c1 = c10 + wy * (c11 - c10)
                outv[s] = c0 + wz * (c1 - c0)
                return c2

            lax.fori_loop(0, NV, body, 0)
            base = base_w + g * CH
            pltpu.sync_copy(outv, out_h.at[pl.ds(base, CH)])

        compute(0, idxa, wxa, wya, wza)
        start(idxa, vala, sema)

        def pair(i, carry):
            compute(2 * i + 1, idxb, wxb, wyb, wzb)
            wait(idxa, vala, sema)
            start(idxb, valb, semb)
            combine(2 * i, vala, wxa, wya, wza)

            @pl.when(i < NPAIR - 1)
            def _():
                compute(2 * i + 2, idxa, wxa, wya, wza)

            wait(idxb, valb, semb)

            @pl.when(i < NPAIR - 1)
            def _():
                start(idxa, vala, sema)

            combine(2 * i + 1, valb, wxb, wyb, wzb)
            return carry

        lax.fori_loop(0, NPAIR, pair, 0)

    return k


_SAMPLE = _build()


def kernel(xyz, t, volume):
    del t
    xs = xyz[:, 0]
    ys = xyz[:, 1]
    zs = xyz[:, 2]
    vol = volume.reshape(-1)
    out = _SAMPLE(xs, ys, zs, vol)
    return out.reshape(-1, 1)
